# baseline (speedup 1.0000x reference)
import functools

import jax
import jax.numpy as jnp
from jax.experimental import pallas as pl

N, D, O, E = 2048, 768, 768, 8
BN = 1024


def _moe_kernel(x_ref, wgt_ref, bg_ref, wet_ref, be_ref, out_ref):
    xb = x_ref[...]
    logits = jax.lax.dot_general(
        xb, wgt_ref[...], (((1,), (0,)), ((), ())),
        preferred_element_type=jnp.float32,
    ) + bg_ref[...]

    iota = jax.lax.broadcasted_iota(jnp.int32, logits.shape, 1)
    big = jnp.int32(E)
    v0 = jnp.max(logits, axis=-1, keepdims=True)
    e0 = jnp.min(jnp.where(logits == v0, iota, big), axis=-1, keepdims=True)
    masked = jnp.where(iota == e0, -jnp.inf, logits)
    v1 = jnp.max(masked, axis=-1, keepdims=True)
    e1 = jnp.min(jnp.where(masked == v1, iota, big), axis=-1, keepdims=True)

    w0 = 1.0 / (1.0 + jnp.exp(v1 - v0))
    w1 = 1.0 - w0
    w_full = jnp.where(iota == e0, w0, 0.0) + jnp.where(iota == e1, w1, 0.0)

    acc = jax.lax.dot_general(
        w_full, be_ref[...], (((1,), (0,)), ((), ())),
        preferred_element_type=jnp.float32,
        precision=jax.lax.Precision.HIGHEST,
    )

    for e in range(E):
        pe = jax.lax.dot_general(
            xb, wet_ref[e], (((1,), (0,)), ((), ())),
            preferred_element_type=jnp.float32,
        )
        acc = acc + w_full[:, e][:, None] * pe
    out_ref[...] = acc


@jax.jit
def kernel(x, Wg, bg, We, be):
    xb = x.astype(jnp.bfloat16)
    wgt = Wg.T.astype(jnp.bfloat16)
    wet = jnp.transpose(We, (0, 2, 1)).astype(jnp.bfloat16)
    bg2 = bg[None, :]
    grid = (N // BN,)
    return pl.pallas_call(
        _moe_kernel,
        grid=grid,
        in_specs=[
            pl.BlockSpec((BN, D), lambda i: (i, 0)),
            pl.BlockSpec((D, E), lambda i: (0, 0)),
            pl.BlockSpec((1, E), lambda i: (0, 0)),
            pl.BlockSpec((E, D, O), lambda i: (0, 0, 0)),
            pl.BlockSpec((E, O), lambda i: (0, 0)),
        ],
        out_specs=pl.BlockSpec((BN, O), lambda i: (i, 0)),
        out_shape=jax.ShapeDtypeStruct((N, O), jnp.float32),
    )(xb, wgt, bg2, wet, be)


# speedup vs baseline: 1.4294x; 1.4294x over previous
import functools

import jax
import jax.numpy as jnp
from jax.experimental import pallas as pl

N, D, O, E = 2048, 768, 768, 8
BN = 1024


def _moe_kernel(x_ref, wgt_ref, bg_ref, wet_ref, be_ref, out_ref):
    xb = x_ref[...].astype(jnp.bfloat16)
    logits = jax.lax.dot_general(
        xb, wgt_ref[...].astype(jnp.bfloat16), (((1,), (0,)), ((), ())),
        preferred_element_type=jnp.float32,
    ) + bg_ref[...]

    iota = jax.lax.broadcasted_iota(jnp.int32, logits.shape, 1)
    big = jnp.int32(E)
    v0 = jnp.max(logits, axis=-1, keepdims=True)
    e0 = jnp.min(jnp.where(logits == v0, iota, big), axis=-1, keepdims=True)
    masked = jnp.where(iota == e0, -jnp.inf, logits)
    v1 = jnp.max(masked, axis=-1, keepdims=True)
    e1 = jnp.min(jnp.where(masked == v1, iota, big), axis=-1, keepdims=True)

    w0 = 1.0 / (1.0 + jnp.exp(v1 - v0))
    w1 = 1.0 - w0
    w_full = jnp.where(iota == e0, w0, 0.0) + jnp.where(iota == e1, w1, 0.0)

    acc = jax.lax.dot_general(
        w_full, be_ref[...], (((1,), (0,)), ((), ())),
        preferred_element_type=jnp.float32,
        precision=jax.lax.Precision.HIGHEST,
    )

    for e in range(E):
        web = wet_ref[e][...].astype(jnp.bfloat16)
        pe = jax.lax.dot_general(
            xb, web, (((1,), (1,)), ((), ())),
            preferred_element_type=jnp.float32,
        )
        acc = acc + w_full[:, e][:, None] * pe
    out_ref[...] = acc


@jax.jit
def kernel(x, Wg, bg, We, be):
    wgt = Wg.T
    bg2 = bg[None, :]
    grid = (N // BN,)
    return pl.pallas_call(
        _moe_kernel,
        grid=grid,
        in_specs=[
            pl.BlockSpec((BN, D), lambda i: (i, 0)),
            pl.BlockSpec((D, E), lambda i: (0, 0)),
            pl.BlockSpec((1, E), lambda i: (0, 0)),
            pl.BlockSpec((E, O, D), lambda i: (0, 0, 0)),
            pl.BlockSpec((E, O), lambda i: (0, 0)),
        ],
        out_specs=pl.BlockSpec((BN, O), lambda i: (i, 0)),
        out_shape=jax.ShapeDtypeStruct((N, O), jnp.float32),
    )(x, wgt, bg2, We, be)
